# speedup vs baseline: 1.1082x; 1.1082x over previous
import jax
import jax.numpy as jnp
from jax import lax
from jax.experimental import pallas as pl
from jax.experimental.pallas import tpu as pltpu
from jax.experimental.pallas import tpu_sc as plsc

N_NODES = 1000000
D = 128
B = 16384
L = 16
NC = 2
NS = 16
NW = NC * NS
ROWS_PER_W = B // NW
CHUNKS = 4
CHUNK_ROWS = ROWS_PER_W // CHUNKS
BLOCKS_PER_CHUNK = CHUNK_ROWS // L

_GATHER_DNUMS = lax.GatherDimensionNumbers(
    offset_dims=(), collapsed_slice_dims=(0,), start_index_map=(0,))


def _splat(vec, lane):
    idx = jnp.full((L, 1), lane, dtype=jnp.int32)
    return lax.gather(vec, idx, _GATHER_DNUMS, (1,),
                      mode=lax.GatherScatterMode.PROMISE_IN_BOUNDS)


def _sc_body(mem_hbm, idx_hbm, td_hbm, w_hbm, b_hbm, out_hbm,
             idx_v, td_v, w_v, b_v, rows_v, gsems, wsems):
    wid = lax.axis_index("s") * NC + lax.axis_index("c")
    base = wid * ROWS_PER_W

    pltpu.sync_copy(idx_hbm.at[wid], idx_v)
    pltpu.sync_copy(td_hbm.at[wid], td_v)
    pltpu.sync_copy(w_hbm, w_v)
    pltpu.sync_copy(b_hbm, b_v)

    gathers = [
        pltpu.async_copy(mem_hbm.at[idx_v.at[k]], rows_v.at[k], gsems[k])
        for k in range(CHUNKS)
    ]

    w_c = [w_v[pl.ds(c * L, L)] for c in range(D // L)]
    b1_c = [b_v[pl.ds(c * L, L)] + 1.0 for c in range(D // L)]

    writes = []
    for k in range(CHUNKS):
        gathers[k].wait()
        rows_k = rows_v.at[k]

        def block_body(j, carry, k=k, rows_k=rows_k):
            tdv = td_v[pl.ds(k * CHUNK_ROWS + j * L, L)]
            for lane in range(L):
                tds = _splat(tdv, lane)
                r = j * L + lane
                for c in range(D // L):
                    sl = pl.ds(c * L, L)
                    rows_k[r, sl] = rows_k[r, sl] * (tds * w_c[c] + b1_c[c])
            return carry

        lax.fori_loop(0, BLOCKS_PER_CHUNK, block_body, 0)
        writes.append(
            pltpu.async_copy(
                rows_k, out_hbm.at[pl.ds(base + k * CHUNK_ROWS, CHUNK_ROWS)],
                wsems[k]))

    for wr in writes:
        wr.wait()


@jax.jit
def _time_embedding_sc(memory, idx3, td2, w, b):
    mesh = plsc.VectorSubcoreMesh(
        core_axis_name="c", subcore_axis_name="s",
        num_cores=NC, num_subcores=NS)
    return pl.kernel(
        _sc_body,
        out_type=jax.ShapeDtypeStruct((B, D), jnp.float32),
        mesh=mesh,
        scratch_types=[
            pltpu.VMEM((CHUNKS, CHUNK_ROWS), jnp.int32),
            pltpu.VMEM((ROWS_PER_W,), jnp.float32),
            pltpu.VMEM((D,), jnp.float32),
            pltpu.VMEM((D,), jnp.float32),
            pltpu.VMEM((CHUNKS, CHUNK_ROWS, D), jnp.float32),
            [pltpu.SemaphoreType.DMA] * CHUNKS,
            [pltpu.SemaphoreType.DMA] * CHUNKS,
        ],
    )(memory, idx3, td2, w, b)


def kernel(memory, source_nodes, timestamps, n_layers, n_neighbors,
           time_diffs, W, b):
    idx3 = source_nodes.reshape(NW, CHUNKS, CHUNK_ROWS)
    td2 = time_diffs.reshape(NW, ROWS_PER_W)
    return _time_embedding_sc(memory, idx3, td2, W[:, 0], b)
